# Initial kernel scaffold; baseline (speedup 1.0000x reference)
import jax
import jax.numpy as jnp
from jax.experimental import pallas as pl


def kernel(hidden_states, norm_w, Wr, expert_bias, Wg, Wu, Wd, Wsg, Wsu, Wsd):
    raise NotImplementedError("write your pallas kernel here")


# speedup vs baseline: 1.9276x; 1.9276x over previous
import functools

import jax
import jax.numpy as jnp
from jax.experimental import pallas as pl
from jax.experimental.pallas import tpu as pltpu

B, S, D = 1, 2048, 768
E, K, F = 8, 2, 512
EPS = 1e-06
SCALE = 2.5
BT = 256
LANES = 128


def _moe_block_kernel(h_ref, nw_ref, wr_ref, bias_ref, wg_ref, wu_ref, wd_ref,
                      wsg_ref, wsu_ref, wsd_ref, out_ref):
    x = h_ref[...]
    var = jnp.mean(x * x, axis=1, keepdims=True)
    normed = x * jax.lax.rsqrt(var + EPS) * nw_ref[...]

    logits = jnp.dot(normed, wr_ref[...], preferred_element_type=jnp.float32)
    scores = jax.nn.sigmoid(logits)
    lane = jax.lax.broadcasted_iota(jnp.int32, (BT, LANES), 1)
    valid = lane < E
    biased = jnp.where(valid, scores + bias_ref[...], -1e30)

    m1 = jnp.max(biased, axis=1, keepdims=True)
    idx1 = jnp.min(jnp.where(biased == m1, lane, LANES), axis=1, keepdims=True)
    oh1 = lane == idx1
    biased2 = jnp.where(oh1, -1e30, biased)
    m2 = jnp.max(biased2, axis=1, keepdims=True)
    idx2 = jnp.min(jnp.where(biased2 == m2, lane, LANES), axis=1, keepdims=True)
    oh2 = lane == idx2

    w1 = jnp.sum(jnp.where(oh1, scores, 0.0), axis=1, keepdims=True)
    w2 = jnp.sum(jnp.where(oh2, scores, 0.0), axis=1, keepdims=True)
    denom = w1 + w2 + 1e-9
    w1 = w1 / denom * SCALE
    w2 = w2 / denom * SCALE
    combine = jnp.where(oh1, w1, 0.0) + jnp.where(oh2, w2, 0.0)

    sg = jnp.dot(normed, wsg_ref[...], preferred_element_type=jnp.float32)
    su = jnp.dot(normed, wsu_ref[...], preferred_element_type=jnp.float32)
    acc = jnp.dot(jax.nn.silu(sg) * su, wsd_ref[...],
                  preferred_element_type=jnp.float32)

    for e in range(E):
        g = jnp.dot(normed, wg_ref[e], preferred_element_type=jnp.float32)
        u = jnp.dot(normed, wu_ref[e], preferred_element_type=jnp.float32)
        ff = jax.nn.silu(g) * u
        ce = jnp.sum(jnp.where(lane == e, combine, 0.0), axis=1, keepdims=True)
        acc = acc + ce * jnp.dot(ff, wd_ref[e], preferred_element_type=jnp.float32)

    out_ref[...] = x + acc


@jax.jit
def kernel(hidden_states, norm_w, Wr, expert_bias, Wg, Wu, Wd, Wsg, Wsu, Wsd):
    b, s, d = hidden_states.shape
    h = hidden_states.reshape(b * s, d)
    T = b * s

    wr_pad = jnp.zeros((d, LANES), jnp.float32).at[:, :E].set(Wr)
    bias_pad = jnp.zeros((1, LANES), jnp.float32).at[0, :E].set(expert_bias)

    grid = (T // BT,)
    out = pl.pallas_call(
        _moe_block_kernel,
        grid=grid,
        in_specs=[
            pl.BlockSpec((BT, d), lambda i: (i, 0)),
            pl.BlockSpec((1, d), lambda i: (0, 0)),
            pl.BlockSpec((d, LANES), lambda i: (0, 0)),
            pl.BlockSpec((1, LANES), lambda i: (0, 0)),
            pl.BlockSpec((E, d, F), lambda i: (0, 0, 0)),
            pl.BlockSpec((E, d, F), lambda i: (0, 0, 0)),
            pl.BlockSpec((E, F, d), lambda i: (0, 0, 0)),
            pl.BlockSpec((d, F), lambda i: (0, 0)),
            pl.BlockSpec((d, F), lambda i: (0, 0)),
            pl.BlockSpec((F, d), lambda i: (0, 0)),
        ],
        out_specs=pl.BlockSpec((BT, d), lambda i: (i, 0)),
        out_shape=jax.ShapeDtypeStruct((T, d), jnp.float32),
        compiler_params=pltpu.CompilerParams(
            dimension_semantics=("arbitrary",),
        ),
    )(h, norm_w.reshape(1, d), wr_pad, bias_pad, Wg, Wu, Wd, Wsg, Wsu, Wsd)

    return out.reshape(b, s, d)
